# baseline (speedup 1.0000x reference)
import jax
import jax.numpy as jnp
from jax.experimental import pallas as pl
from jax.experimental.pallas import tpu as pltpu

_BM = 4096
_NSPLIT = 2


def _linear_clip_kernel(x_ref, w_ref, b_ref, o_ref):
    acc = jax.lax.dot_general(
        x_ref[:], w_ref[:],
        dimension_numbers=(((1,), (1,)), ((), ())),
        preferred_element_type=jnp.float32,
    )
    o_ref[:] = jnp.clip(acc + b_ref[:], 0.0, 1.0)


def kernel(x, weight, bias):
    m, k = x.shape
    n = weight.shape[0]
    bn = n // _NSPLIT
    bias2d = bias.reshape(1, n)
    return pl.pallas_call(
        _linear_clip_kernel,
        grid=(m // _BM, _NSPLIT),
        in_specs=[
            pl.BlockSpec((_BM, k), lambda i, j: (i, 0)),
            pl.BlockSpec((bn, k), lambda i, j: (j, 0)),
            pl.BlockSpec((1, bn), lambda i, j: (0, j)),
        ],
        out_specs=pl.BlockSpec((_BM, bn), lambda i, j: (i, j)),
        out_shape=jax.ShapeDtypeStruct((m, n), jnp.float32),
        compiler_params=pltpu.CompilerParams(
            dimension_semantics=("parallel", "parallel"),
        ),
    )(x, weight, bias2d)


# speedup vs baseline: 1.4247x; 1.4247x over previous
import jax
import jax.numpy as jnp
from jax.experimental import pallas as pl
from jax.experimental.pallas import tpu as pltpu

_BT = 2048
_NBUF = 3


def _make_body(m, k, n):
    t_total = m // _BT

    def body(x_hbm, w_ref, b_ref, o_hbm, x_vmem, o_vmem, lsem, ssem):
        def load(t):
            b = t % _NBUF
            return pltpu.make_async_copy(
                x_hbm.at[pl.ds(t * _BT, _BT), :], x_vmem.at[b], lsem.at[b])

        def store(t):
            b = t % _NBUF
            return pltpu.make_async_copy(
                o_vmem.at[b], o_hbm.at[pl.ds(t * _BT, _BT), :], ssem.at[b])

        for t in range(min(_NBUF, t_total)):
            load(t).start()
        for t in range(t_total):
            b = t % _NBUF
            load(t).wait()
            if t >= _NBUF:
                store(t - _NBUF).wait()
            acc = jax.lax.dot_general(
                x_vmem[b], w_ref[:],
                dimension_numbers=(((1,), (1,)), ((), ())),
                preferred_element_type=jnp.float32,
            )
            o_vmem[b] = jnp.clip(acc + b_ref[:], 0.0, 1.0)
            store(t).start()
            if t + _NBUF < t_total:
                load(t + _NBUF).start()
        for t in range(max(0, t_total - _NBUF), t_total):
            store(t).wait()

    return body


def kernel(x, weight, bias):
    m, k = x.shape
    n = weight.shape[0]
    bias2d = bias.reshape(1, n)
    return pl.pallas_call(
        _make_body(m, k, n),
        in_specs=[
            pl.BlockSpec(memory_space=pl.ANY),
            pl.BlockSpec((n, k), lambda: (0, 0)),
            pl.BlockSpec((1, n), lambda: (0, 0)),
        ],
        out_specs=pl.BlockSpec(memory_space=pl.ANY),
        out_shape=jax.ShapeDtypeStruct((m, n), jnp.float32),
        scratch_shapes=[
            pltpu.VMEM((_NBUF, _BT, k), jnp.float32),
            pltpu.VMEM((_NBUF, _BT, n), jnp.float32),
            pltpu.SemaphoreType.DMA((_NBUF,)),
            pltpu.SemaphoreType.DMA((_NBUF,)),
        ],
    )(x, weight, bias2d)


# speedup vs baseline: 1.4632x; 1.0270x over previous
import jax
import jax.numpy as jnp
from jax.experimental import pallas as pl
from jax.experimental.pallas import tpu as pltpu

_BT = 1024
_NBUF = 6


def _make_body(m, k, n):
    t_total = m // _BT

    def body(x_hbm, w_ref, b_ref, o_hbm, x_vmem, o_vmem, lsem, ssem):
        def load(t):
            b = t % _NBUF
            return pltpu.make_async_copy(
                x_hbm.at[pl.ds(t * _BT, _BT), :], x_vmem.at[b], lsem.at[b])

        def store(t):
            b = t % _NBUF
            return pltpu.make_async_copy(
                o_vmem.at[b], o_hbm.at[pl.ds(t * _BT, _BT), :], ssem.at[b])

        for t in range(min(_NBUF, t_total)):
            load(t).start()
        for t in range(t_total):
            b = t % _NBUF
            load(t).wait()
            if t >= _NBUF:
                store(t - _NBUF).wait()
            acc = jax.lax.dot_general(
                x_vmem[b], w_ref[:],
                dimension_numbers=(((1,), (1,)), ((), ())),
                preferred_element_type=jnp.float32,
            )
            o_vmem[b] = jnp.clip(acc + b_ref[:], 0.0, 1.0)
            if t + _NBUF < t_total:
                load(t + _NBUF).start()
            store(t).start()
        for t in range(max(0, t_total - _NBUF), t_total):
            store(t).wait()

    return body


def kernel(x, weight, bias):
    m, k = x.shape
    n = weight.shape[0]
    bias2d = bias.reshape(1, n)
    return pl.pallas_call(
        _make_body(m, k, n),
        in_specs=[
            pl.BlockSpec(memory_space=pl.ANY),
            pl.BlockSpec((n, k), lambda: (0, 0)),
            pl.BlockSpec((1, n), lambda: (0, 0)),
        ],
        out_specs=pl.BlockSpec(memory_space=pl.ANY),
        out_shape=jax.ShapeDtypeStruct((m, n), jnp.float32),
        scratch_shapes=[
            pltpu.VMEM((_NBUF, _BT, k), jnp.float32),
            pltpu.VMEM((_NBUF, _BT, n), jnp.float32),
            pltpu.SemaphoreType.DMA((_NBUF,)),
            pltpu.SemaphoreType.DMA((_NBUF,)),
        ],
    )(x, weight, bias2d)
